# baseline (speedup 1.0000x reference)
import functools

import jax
import jax.numpy as jnp
from jax.experimental import pallas as pl

_VOCAB = 1000
_ROWS_PER_BLOCK = 2048


def _hist_body(x_ref, wc_ref, out_ref):
    i = pl.program_id(0)
    x = x_ref[...]
    idx = jnp.argmax(x, axis=1).astype(jnp.int32)
    bins = jax.lax.broadcasted_iota(jnp.int32, (x.shape[0], _VOCAB), 1)
    onehot = (idx[:, None] == bins).astype(jnp.float32)
    hist = jnp.sum(onehot, axis=0)

    @pl.when(i == 0)
    def _():
        out_ref[...] = wc_ref[...]

    out_ref[0, :] += hist


def kernel(utterances, word_counts):
    batch, agents, vocab = utterances.shape
    rows = utterances.reshape(batch * agents, vocab)
    nblk = (batch * agents) // _ROWS_PER_BLOCK
    out = pl.pallas_call(
        _hist_body,
        grid=(nblk,),
        in_specs=[
            pl.BlockSpec((_ROWS_PER_BLOCK, vocab), lambda i: (i, 0)),
            pl.BlockSpec((1, vocab), lambda i: (0, 0)),
        ],
        out_specs=pl.BlockSpec((1, vocab), lambda i: (0, 0)),
        out_shape=jax.ShapeDtypeStruct((1, vocab), jnp.float32),
    )(rows, word_counts.reshape(1, vocab))
    return out.reshape(vocab)
